# baseline (speedup 1.0000x reference)
import functools

import jax
import jax.numpy as jnp
from jax import lax
from jax.experimental import pallas as pl
from jax.experimental.pallas import tpu as pltpu
from jax.experimental.pallas import tpu_sc as plsc

N = 10000
E = 320000
NPAD = 10240
NW = 32
EPW = E // NW
CHUNK = 128
NCH = 80
KW = NCH * CHUNK
ZROWS = NPAD // 16
NBUF = 5
D = 3


def _make_spmm(cp):
    mesh = plsc.VectorSubcoreMesh(core_axis_name="c", subcore_axis_name="s")

    @functools.partial(
        pl.kernel,
        mesh=mesh,
        compiler_params=pltpu.CompilerParams(use_tc_tiling_on_sc=False),
        out_type=jax.ShapeDtypeStruct((2 * NPAD, cp), jnp.float32),
        scratch_types=[
            pltpu.VMEM((NCH, CHUNK), jnp.int32),
            pltpu.VMEM((NCH, CHUNK), jnp.int32),
            [pltpu.VMEM((CHUNK, cp), jnp.float32)] * NBUF,
            pltpu.VMEM_SHARED((NPAD, cp), jnp.float32),
            [pltpu.SemaphoreType.DMA] * NBUF,
            [pltpu.SemaphoreType.DMA] * NBUF,
        ],
    )
    def spmm(ys_hbm, src_hbm, dst_hbm, out_hbm, src_v, dst_v, bufs,
             acc, gsems, ssems):
        c = lax.axis_index("c")
        s = lax.axis_index("s")
        w = c * 16 + s
        base = s * ZROWS

        pltpu.async_copy(src_hbm.at[w], src_v, ssems[0])
        pltpu.async_copy(dst_hbm.at[w], dst_v, ssems[1])

        zero16 = jnp.zeros((16,), jnp.float32)

        def zrow(i, carry):
            for j in range(cp // 16):
                bufs[0][i, pl.ds(j * 16, 16)] = zero16
            return carry

        lax.fori_loop(0, CHUNK, zrow, 0)
        for t in range(ZROWS // CHUNK):
            pltpu.async_copy(bufs[0],
                             acc.at[pl.ds(base + t * CHUNK, CHUNK)],
                             gsems[t % NBUF])
        for t in range(ZROWS // CHUNK):
            pltpu.make_async_copy(bufs[0],
                                  acc.at[pl.ds(base + t * CHUNK, CHUNK)],
                                  gsems[t % NBUF]).wait()
        pltpu.make_async_copy(src_hbm.at[w], src_v, ssems[0]).wait()
        pltpu.make_async_copy(dst_hbm.at[w], dst_v, ssems[1]).wait()
        plsc.subcore_barrier()

        def g_start(j, b):
            pltpu.async_copy(ys_hbm.at[src_v.at[j]], bufs[b], gsems[b])

        def g_wait(j, b):
            pltpu.make_async_copy(ys_hbm.at[src_v.at[j]], bufs[b],
                                  gsems[b]).wait()

        def s_start(j, b):
            pltpu.async_copy(bufs[b], acc.at[dst_v.at[j]], ssems[b],
                             add=True)

        def s_wait(j, b):
            pltpu.make_async_copy(bufs[b], acc.at[dst_v.at[j]],
                                  ssems[b]).wait()

        for b in range(D):
            g_start(b, b)

        def outer(g, carry):
            j0 = g * NBUF
            for u in range(NBUF):
                j = j0 + u
                g_wait(j, u)
                s_start(j, u)
                k = j + D
                bk = (u + D) % NBUF

                @pl.when(k < NCH)
                def _():
                    @pl.when(k >= NBUF)
                    def _():
                        s_wait(k - NBUF, bk)

                    g_start(k, bk)

            return carry

        lax.fori_loop(0, NCH // NBUF, outer, 0)
        for u in range(NBUF):
            s_wait(NCH - NBUF + u, u)
        plsc.subcore_barrier()

        pltpu.sync_copy(acc.at[pl.ds(base, ZROWS)],
                        out_hbm.at[pl.ds(c * NPAD + base, ZROWS)])

    return spmm


_spmm16 = _make_spmm(16)
_spmm80 = _make_spmm(80)


R = 2000


def _tc(body, in_kinds, out_widths):
    in_specs = []
    for kind, p in in_kinds:
        if kind == "S":
            in_specs.append(pl.BlockSpec((2, R, p), lambda i: (0, i, 0)))
        elif kind == "r":
            in_specs.append(pl.BlockSpec((R, p), lambda i: (i, 0)))
        else:
            in_specs.append(pl.BlockSpec(p, lambda i: (0,) * len(p)))
    outs = [w if isinstance(w, tuple) else (w, N) for w in out_widths]
    return pl.pallas_call(
        body,
        grid=(N // R,),
        in_specs=in_specs,
        out_specs=[pl.BlockSpec((R, w), lambda i: (i, 0))
                   for w, _ in outs],
        out_shape=[jax.ShapeDtypeStruct((rows, w), jnp.float32)
                   for w, rows in outs],
    )


_DOT = functools.partial(jnp.dot, precision=lax.Precision.HIGHEST,
                         preferred_element_type=jnp.float32)


def _comb(Sref, lo, hi, dinv, y, b):
    S = Sref[0, :, lo:hi] + Sref[1, :, lo:hi]
    return dinv * S + 2.0 * dinv * dinv * y + b


def _padw(v, cp):
    return jnp.pad(v, ((0, 0), (0, cp - v.shape[1])))


def _prep_body(degS, x, W1, W1_2, dinv_o, xw1_o, xw2_o,
               ysa_o, ysb_o, ysc_o):
    deg = degS[0, :, 0:1] + degS[1, :, 0:1] + 2.0
    dinv = 1.0 / jnp.sqrt(deg)
    dinv_o[...] = dinv
    xw1 = _DOT(x[...], W1[...])
    xw2 = _DOT(x[...], W1_2[...])
    xw1_o[...] = xw1
    xw2_o[...] = xw2
    ysa_o[...] = dinv * xw1[:, 0:80]
    ysb_o[...] = dinv * xw2[:, 0:80]
    ysc_o[...] = _padw(
        dinv * jnp.concatenate([xw1[:, 80:100], xw2[:, 80:100]], axis=1), 80)


def _dense12_body(S1a, S1b, S1c, xw1, xw2, dinv_r, b1, b1_2, W2, W2_2,
                  y2m_o, y2f_o, ys2_o):
    dinv = dinv_r[...]
    Sa = S1a[0, :, :] + S1a[1, :, :]
    Sb = S1b[0, :, :] + S1b[1, :, :]
    Sc = S1c[0, :, :] + S1c[1, :, :]
    S1m = jnp.concatenate([Sa, Sc[:, 0:20]], axis=1)
    S1f = jnp.concatenate([Sb, Sc[:, 20:40]], axis=1)
    d2 = 2.0 * dinv * dinv
    h1 = jax.nn.relu(dinv * S1m + d2 * xw1[...] + b1[...])
    h2 = jax.nn.relu(dinv * S1f + d2 * xw2[...] + b1_2[...])
    y2m = _DOT(h1, W2[...])
    y2f = _DOT(h2, W2_2[...])
    y2m_o[...] = y2m
    y2f_o[...] = y2f
    ys2_o[...] = _padw(dinv * jnp.concatenate([y2m, y2f], axis=1), 16)


def _dense3_body(S2, y2m, y2f, dinv_r, m, f, b2, b2_2,
                 c2m_o, c2f_o, ys3_o):
    dinv = dinv_r[...]
    c2m = _comb(S2, 0, 1, dinv, y2m[...], b2[...])
    c2f = _comb(S2, 1, 2, dinv, y2f[...], b2_2[...])
    c2m_o[...] = c2m
    c2f_o[...] = c2f
    ys3_o[...] = _padw(
        dinv * jnp.concatenate([c2m, c2f, m[...], f[...]], axis=1), 16)


def _dense4_body(S3, c2m, c2f, m, f, dinv_r, W2m, W2f, b2m, b2f,
                 W2m_1, W2f_1, y4m_o, y4f_o, ys4_o):
    dinv = dinv_r[...]
    t_c2m = S3[0, :, 0:1] + S3[1, :, 0:1]
    t_c2f = S3[0, :, 1:2] + S3[1, :, 1:2]
    t_m = S3[0, :, 2:3] + S3[1, :, 2:3]
    t_f = S3[0, :, 3:4] + S3[1, :, 3:4]
    S3m = _DOT(t_c2m, W2m[0:1, :]) + _DOT(t_m, W2m[1:2, :])
    S3f = _DOT(t_c2f, W2f[0:1, :]) + _DOT(t_f, W2f[1:2, :])
    y3m = _DOT(c2m[...], W2m[0:1, :]) + _DOT(m[...], W2m[1:2, :])
    y3f = _DOT(c2f[...], W2f[0:1, :]) + _DOT(f[...], W2f[1:2, :])
    d2 = 2.0 * dinv * dinv
    hm2 = jax.nn.relu(dinv * S3m + d2 * y3m + b2m[...])
    hf2 = jax.nn.relu(dinv * S3f + d2 * y3f + b2f[...])
    y4m = _DOT(hm2, W2m_1[...])
    y4f = _DOT(hf2, W2f_1[...])
    y4m_o[...] = y4m
    y4f_o[...] = y4f
    ys4_o[...] = _padw(dinv * jnp.concatenate([y4m, y4f], axis=1), 16)


def _dense5_body(S4, y4m, y4f, dinv_r, b2m_1, b2f_1, WA,
                 hmbr_o, hfbr_o, y5_o, ys5_o):
    dinv = dinv_r[...]
    hm_br = _comb(S4, 0, 1, dinv, y4m[...], b2m_1[...])
    hf_br = _comb(S4, 1, 2, dinv, y4f[...], b2f_1[...])
    hmbr_o[...] = hm_br
    hfbr_o[...] = hf_br
    hcat = jnp.concatenate([jax.nn.relu(hm_br), jax.nn.relu(hf_br)], axis=1)
    y5 = _DOT(hcat, WA[...])
    y5_o[...] = y5
    ys5_o[...] = _padw(dinv * y5, 16)


def _dense6_body(S5, y5, dinv_r, bA, WA_1, y6_o, ys6_o):
    dinv = dinv_r[...]
    hA = jax.nn.relu(_comb(S5, 0, 10, dinv, y5[...], bA[...]))
    y6 = _DOT(hA, WA_1[...])
    y6_o[...] = y6
    ys6_o[...] = _padw(dinv * y6, 16)


def _dense7_body(S6, y6, dinv_r, bA_1, h_o):
    dinv = dinv_r[...]
    h_o[...] = _comb(S6, 0, 1, dinv, y6[...], bA_1[...])


def kernel(x, edge_index, edge_weight, m, f, W1, b1, W1_2, b1_2, W2, b2,
           W2_2, b2_2, W2m, b2m, W2m_1, b2m_1, W2f, b2f, W2f_1, b2f_1,
           WA, bA, WA_1, bA_1):
    pad_idx = jnp.arange(KW - EPW, dtype=jnp.int32) % 16
    pad_src = jnp.broadcast_to(pad_idx, (NW, KW - EPW))
    pad_dst = jnp.broadcast_to(N + pad_idx, (NW, KW - EPW))
    srcw = jnp.concatenate([edge_index[0].reshape(NW, EPW), pad_src], axis=1)
    dstw = jnp.concatenate([edge_index[1].reshape(NW, EPW), pad_dst], axis=1)
    srcw = srcw.reshape(NW, NCH, CHUNK)
    dstw = dstw.reshape(NW, NCH, CHUNK)

    b1r = b1.reshape(1, -1)
    b1_2r = b1_2.reshape(1, -1)
    b2r = b2.reshape(1, -1)
    b2_2r = b2_2.reshape(1, -1)
    b2mr = b2m.reshape(1, -1)
    b2fr = b2f.reshape(1, -1)
    b2m_1r = b2m_1.reshape(1, -1)
    b2f_1r = b2f_1.reshape(1, -1)
    bAr = bA.reshape(1, -1)
    bA_1r = bA_1.reshape(1, -1)

    ones_t = jnp.ones((NPAD, 16), jnp.float32)
    degS = _spmm16(ones_t, srcw, dstw).reshape(2, NPAD, 16)

    dinv, xw1, xw2, ysa, ysb, ysc = _tc(
        _prep_body,
        [("S", 16), ("r", 128), ("w", (128, 100)), ("w", (128, 100))],
        [1, 100, 100, (80, NPAD), (80, NPAD), (80, NPAD)],
    )(degS, x, W1, W1_2)
    S1a = _spmm80(ysa, srcw, dstw).reshape(2, NPAD, 80)
    S1b = _spmm80(ysb, srcw, dstw).reshape(2, NPAD, 80)
    S1c = _spmm80(ysc, srcw, dstw).reshape(2, NPAD, 80)

    y2m, y2f, ys2c = _tc(
        _dense12_body,
        [("S", 80), ("S", 80), ("S", 80), ("r", 100), ("r", 100), ("r", 1),
         ("w", (1, 100)), ("w", (1, 100)), ("w", (100, 1)), ("w", (100, 1))],
        [1, 1, (16, NPAD)],
    )(S1a, S1b, S1c, xw1, xw2, dinv, b1r, b1_2r, W2, W2_2)
    S2 = _spmm16(ys2c, srcw, dstw).reshape(2, NPAD, 16)

    c2m, c2f, ys3 = _tc(
        _dense3_body,
        [("S", 16), ("r", 1), ("r", 1), ("r", 1), ("r", 1), ("r", 1),
         ("w", (1, 1)), ("w", (1, 1))],
        [1, 1, (16, NPAD)],
    )(S2, y2m, y2f, dinv, m, f, b2r, b2_2r)
    S3 = _spmm16(ys3, srcw, dstw).reshape(2, NPAD, 16)

    y4m, y4f, ys4 = _tc(
        _dense4_body,
        [("S", 16), ("r", 1), ("r", 1), ("r", 1), ("r", 1), ("r", 1),
         ("w", (2, 10)), ("w", (2, 10)), ("w", (1, 10)), ("w", (1, 10)),
         ("w", (10, 1)), ("w", (10, 1))],
        [1, 1, (16, NPAD)],
    )(S3, c2m, c2f, m, f, dinv, W2m, W2f, b2mr, b2fr, W2m_1, W2f_1)
    S4 = _spmm16(ys4, srcw, dstw).reshape(2, NPAD, 16)

    hm_br, hf_br, y5, ys5 = _tc(
        _dense5_body,
        [("S", 16), ("r", 1), ("r", 1), ("r", 1),
         ("w", (1, 1)), ("w", (1, 1)), ("w", (2, 10))],
        [1, 1, 10, (16, NPAD)],
    )(S4, y4m, y4f, dinv, b2m_1r, b2f_1r, WA)
    S5 = _spmm16(ys5, srcw, dstw).reshape(2, NPAD, 16)

    y6, ys6 = _tc(
        _dense6_body,
        [("S", 16), ("r", 10), ("r", 1), ("w", (1, 10)), ("w", (10, 1))],
        [1, (16, NPAD)],
    )(S5, y5, dinv, bAr, WA_1)
    S6 = _spmm16(ys6, srcw, dstw).reshape(2, NPAD, 16)

    (h,) = _tc(
        _dense7_body,
        [("S", 16), ("r", 1), ("r", 1), ("w", (1, 1))],
        [1],
    )(S6, y6, dinv, bA_1r)

    return (h, hm_br, hf_br)
